# Initial kernel scaffold; baseline (speedup 1.0000x reference)
import functools

import jax
import jax.numpy as jnp
from jax import lax
from jax.experimental import pallas as pl
from jax.experimental.pallas import tpu as pltpu
from jax.experimental.pallas import tpu_sc as plsc

N_CT = 6000
M_CT = 4000
NT = N_CT + M_CT
D = 128
E = 320000
LAYERS = 3

NW = 32
CHUNK = 128
EPAD = 327680
CPT = EPAD // (NW * CHUNK)
WPAD = NT + 8
SINK = NT + 8
APAD = 10016
RPT = APAD // 16

_mesh = plsc.VectorSubcoreMesh(core_axis_name="c", subcore_axis_name="s")


@functools.partial(
    pl.kernel,
    out_type=jax.ShapeDtypeStruct((2, APAD, D), jnp.float32),
    mesh=_mesh,
    scratch_types=[
        pltpu.VMEM((CPT, CHUNK), jnp.int32),
        pltpu.VMEM((CPT, CHUNK), jnp.int32),
        pltpu.VMEM((CHUNK, D), jnp.float32),
        pltpu.VMEM_SHARED((APAD, D), jnp.float32),
        pltpu.SemaphoreType.DMA,
    ],
)
def _spmm(w_hbm, col_hbm, row_hbm, zero_hbm, out_hbm,
          colv, rowv, buf, acc, sem):
    c = lax.axis_index("c")
    s = lax.axis_index("s")
    wid = s * 2 + c

    pltpu.sync_copy(col_hbm.at[wid], colv)
    pltpu.sync_copy(row_hbm.at[wid], rowv)
    pltpu.sync_copy(zero_hbm, acc.at[pl.ds(s * RPT, RPT)])
    plsc.subcore_barrier()

    def body(j, carry):
        pltpu.async_copy(w_hbm.at[colv.at[j]], buf, sem).wait()
        pltpu.sync_copy(buf, acc.at[rowv.at[j]], add=True)
        return carry

    lax.fori_loop(0, CPT, body, 0)

    plsc.subcore_barrier()
    pltpu.sync_copy(acc.at[pl.ds(s * RPT, RPT)],
                    out_hbm.at[c, pl.ds(s * RPT, RPT)])


def kernel(all_N_emb, all_M_emb, edge_index):
    row = edge_index[0]
    col = edge_index[1]
    deg = jnp.zeros((NT,), jnp.float32).at[row].add(1.0)
    d = jnp.power(deg + 1e-10, -0.5)
    d2 = d * d

    ego = jnp.concatenate([all_N_emb, all_M_emb], axis=0)

    pad = EPAD - E
    colp = jnp.concatenate([col, jnp.full((pad,), NT, jnp.int32)])
    rowp = jnp.concatenate([row, jnp.full((pad,), SINK, jnp.int32)])
    colp = colp.reshape(NW, CPT, CHUNK)
    rowp = rowp.reshape(NW, CPT, CHUNK)
    zero_blk = jnp.zeros((RPT, D), jnp.float32)
    zpad = jnp.zeros((WPAD - NT, D), jnp.float32)

    acc_mean = ego
    w = jnp.concatenate([d[:, None] * ego, zpad], axis=0)
    for layer in range(LAYERS):
        parts = _spmm(w, colp, rowp, zero_blk)
        ssum = parts[0, :NT] + parts[1, :NT]
        acc_mean = acc_mean + d[:, None] * ssum
        if layer + 1 < LAYERS:
            w = jnp.concatenate([d2[:, None] * ssum, zpad], axis=0)
    mean = acc_mean * 0.25
    return mean[:N_CT], mean[N_CT:NT]


# speedup vs baseline: 4.8915x; 4.8915x over previous
import functools

import jax
import jax.numpy as jnp
from jax import lax
from jax.experimental import pallas as pl
from jax.experimental.pallas import tpu as pltpu
from jax.experimental.pallas import tpu_sc as plsc

N_CT = 6000
M_CT = 4000
NT = N_CT + M_CT
D = 128
E = 320000
LAYERS = 3

NW = 32
CHUNK = 128
EPAD = 327680
CPT = EPAD // (NW * CHUNK)
WPAD = NT + 8
SINK = NT + 8
APAD = 10112
RPT = APAD // 16

_mesh = plsc.VectorSubcoreMesh(core_axis_name="c", subcore_axis_name="s")


@functools.partial(
    pl.kernel,
    out_type=jax.ShapeDtypeStruct((2, APAD, D), jnp.float32),
    mesh=_mesh,
    scratch_types=[
        pltpu.VMEM((CPT, CHUNK), jnp.int32),
        pltpu.VMEM((CPT, CHUNK), jnp.int32),
        pltpu.VMEM((CHUNK, D), jnp.float32),
        pltpu.VMEM_SHARED((APAD, D), jnp.float32),
        pltpu.SemaphoreType.DMA,
    ],
)
def _spmm(w_hbm, col_hbm, row_hbm, zero_hbm, out_hbm,
          colv, rowv, buf, acc, sem):
    c = lax.axis_index("c")
    s = lax.axis_index("s")
    wid = s * 2 + c

    pltpu.sync_copy(col_hbm.at[wid], colv)
    pltpu.sync_copy(row_hbm.at[wid], rowv)
    pltpu.sync_copy(zero_hbm, acc.at[pl.ds(s * RPT, RPT)])
    plsc.subcore_barrier()

    def body(j, carry):
        pltpu.async_copy(w_hbm.at[colv.at[j]], buf, sem).wait()
        pltpu.sync_copy(buf, acc.at[rowv.at[j]], add=True)
        return carry

    lax.fori_loop(0, CPT, body, 0)

    plsc.subcore_barrier()
    pltpu.sync_copy(acc.at[pl.ds(s * RPT, RPT)],
                    out_hbm.at[c, pl.ds(s * RPT, RPT)])


def kernel(all_N_emb, all_M_emb, edge_index):
    row = edge_index[0]
    col = edge_index[1]
    deg = jnp.zeros((NT,), jnp.float32).at[row].add(1.0)
    d = jnp.power(deg + 1e-10, -0.5)
    d2 = d * d

    ego = jnp.concatenate([all_N_emb, all_M_emb], axis=0)

    pad = EPAD - E
    colp = jnp.concatenate([col, jnp.full((pad,), NT, jnp.int32)])
    rowp = jnp.concatenate([row, jnp.full((pad,), SINK, jnp.int32)])
    colp = colp.reshape(NW, CPT, CHUNK)
    rowp = rowp.reshape(NW, CPT, CHUNK)
    zero_blk = jnp.zeros((RPT, D), jnp.float32)
    zpad = jnp.zeros((WPAD - NT, D), jnp.float32)

    acc_mean = ego
    w = jnp.concatenate([d[:, None] * ego, zpad], axis=0)
    for layer in range(LAYERS):
        parts = _spmm(w, colp, rowp, zero_blk)
        ssum = parts[0, :NT] + parts[1, :NT]
        acc_mean = acc_mean + d[:, None] * ssum
        if layer + 1 < LAYERS:
            w = jnp.concatenate([d2[:, None] * ssum, zpad], axis=0)
    mean = acc_mean * 0.25
    return mean[:N_CT], mean[N_CT:NT]


# speedup vs baseline: 14.0234x; 2.8669x over previous
import functools

import jax
import jax.numpy as jnp
from jax import lax
from jax.experimental import pallas as pl
from jax.experimental.pallas import tpu as pltpu
from jax.experimental.pallas import tpu_sc as plsc

N_CT = 6000
M_CT = 4000
NT = N_CT + M_CT
D = 128
E = 320000
LAYERS = 3

NW = 32
CHUNK = 80
CPT = 125
HALF = 64
WPAD = NT + 8
APAD = 10112
RPT = APAD // 16
PAIRS = CPT // 2

_mesh = plsc.VectorSubcoreMesh(core_axis_name="c", subcore_axis_name="s")


@functools.partial(
    pl.kernel,
    out_type=jax.ShapeDtypeStruct((2, APAD, HALF), jnp.float32),
    mesh=_mesh,
    compiler_params=pltpu.CompilerParams(use_tc_tiling_on_sc=False),
    scratch_types=[
        pltpu.VMEM((CPT, CHUNK), jnp.int32),
        pltpu.VMEM((CPT, CHUNK), jnp.int32),
        pltpu.VMEM((CHUNK, HALF), jnp.float32),
        pltpu.VMEM((CHUNK, HALF), jnp.float32),
        pltpu.VMEM_SHARED((APAD, HALF), jnp.float32),
        pltpu.SemaphoreType.DMA,
        pltpu.SemaphoreType.DMA,
    ],
)
def _spmm(w_hbm, col_hbm, row_hbm, zero_hbm, out_hbm,
          colv, rowv, bufa, bufb, acc, sema, semb):
    c = lax.axis_index("c")
    s = lax.axis_index("s")
    wid = s * 2 + c

    pltpu.sync_copy(col_hbm.at[wid], colv)
    pltpu.sync_copy(row_hbm.at[wid], rowv)
    pltpu.sync_copy(zero_hbm, acc.at[pl.ds(s * RPT, RPT)])
    plsc.subcore_barrier()

    pltpu.async_copy(w_hbm.at[colv.at[0]], bufa, sema)

    def body(jj, carry):
        j0 = 2 * jj
        j1 = j0 + 1
        pltpu.async_copy(w_hbm.at[colv.at[j1]], bufb, semb)
        pltpu.make_async_copy(w_hbm.at[colv.at[j0]], bufa, sema).wait()
        pltpu.sync_copy(bufa, acc.at[rowv.at[j0]], add=True)

        @pl.when(j0 + 2 < CPT)
        def _():
            pltpu.async_copy(w_hbm.at[colv.at[j0 + 2]], bufa, sema)

        pltpu.make_async_copy(w_hbm.at[colv.at[j1]], bufb, semb).wait()
        pltpu.sync_copy(bufb, acc.at[rowv.at[j1]], add=True)
        return carry

    lax.fori_loop(0, PAIRS, body, 0)

    if CPT % 2 == 1:
        j = CPT - 1
        pltpu.make_async_copy(w_hbm.at[colv.at[j]], bufa, sema).wait()
        pltpu.sync_copy(bufa, acc.at[rowv.at[j]], add=True)

    plsc.subcore_barrier()
    pltpu.sync_copy(acc.at[pl.ds(s * RPT, RPT)],
                    out_hbm.at[c, pl.ds(s * RPT, RPT)])


@functools.partial(
    pl.kernel,
    out_type=jax.ShapeDtypeStruct((2, APAD, 16), jnp.float32),
    mesh=_mesh,
    compiler_params=pltpu.CompilerParams(use_tc_tiling_on_sc=False),
    scratch_types=[
        pltpu.VMEM((CPT, CHUNK), jnp.int32),
        pltpu.VMEM((CHUNK, 16), jnp.float32),
        pltpu.VMEM_SHARED((APAD, 16), jnp.float32),
    ],
)
def _degree(row_hbm, ones_hbm, zero_hbm, out_hbm, rowv, onesv, acc):
    c = lax.axis_index("c")
    s = lax.axis_index("s")
    wid = s * 2 + c

    pltpu.sync_copy(row_hbm.at[wid], rowv)
    pltpu.sync_copy(ones_hbm, onesv)
    pltpu.sync_copy(zero_hbm, acc.at[pl.ds(s * RPT, RPT)])
    plsc.subcore_barrier()

    def body(j, carry):
        pltpu.sync_copy(onesv, acc.at[rowv.at[j]], add=True)
        return carry

    lax.fori_loop(0, CPT, body, 0)

    plsc.subcore_barrier()
    pltpu.sync_copy(acc.at[pl.ds(s * RPT, RPT)],
                    out_hbm.at[c, pl.ds(s * RPT, RPT)])


def kernel(all_N_emb, all_M_emb, edge_index):
    colp = edge_index[1].reshape(NW, CPT, CHUNK)
    rowp = edge_index[0].reshape(NW, CPT, CHUNK)
    zero_blk = jnp.zeros((RPT, HALF), jnp.float32)
    zpad = jnp.zeros((WPAD - NT, HALF), jnp.float32)

    dparts = _degree(rowp, jnp.ones((CHUNK, 16), jnp.float32),
                     jnp.zeros((RPT, 16), jnp.float32))
    deg = dparts[0, :NT, 0] + dparts[1, :NT, 0]
    d = jnp.power(deg + 1e-10, -0.5)
    d2 = d * d

    ego = jnp.concatenate([all_N_emb, all_M_emb], axis=0)

    acc_mean = ego
    w0 = d[:, None] * ego
    whalves = [jnp.concatenate([w0[:, :HALF], zpad], axis=0),
               jnp.concatenate([w0[:, HALF:], zpad], axis=0)]
    for layer in range(LAYERS):
        shalves = []
        for h in range(2):
            parts = _spmm(whalves[h], colp, rowp, zero_blk)
            shalves.append(parts[0, :NT] + parts[1, :NT])
        ssum = jnp.concatenate(shalves, axis=1)
        acc_mean = acc_mean + d[:, None] * ssum
        if layer + 1 < LAYERS:
            whalves = [jnp.concatenate([d2[:, None] * shalves[0], zpad], axis=0),
                       jnp.concatenate([d2[:, None] * shalves[1], zpad], axis=0)]
    mean = acc_mean * 0.25
    return mean[:N_CT], mean[N_CT:NT]
